# baseline (speedup 1.0000x reference)
import functools

import jax
import jax.numpy as jnp
from jax import lax
from jax.experimental import pallas as pl
from jax.experimental.pallas import tpu as pltpu
from jax.experimental.pallas import tpu_sc as plsc

B = 16384
S = 50
E = 32
HID = 32
NCLS = 10

NC = 2
NS = 16
NW = NC * NS
BPW = B // NW
NBUF = 8

_mesh = plsc.VectorSubcoreMesh(
    core_axis_name="c", subcore_axis_name="s", num_cores=NC, num_subcores=NS
)

_IOTA16 = tuple(range(16))


@functools.partial(
    pl.kernel,
    out_type=jax.ShapeDtypeStruct((B, E), jnp.float32),
    mesh=_mesh,
    scratch_types=[
        pltpu.VMEM((BPW, S), jnp.int32),
        pltpu.VMEM((BPW, S), jnp.int32),
        pltpu.VMEM((NBUF, S, 128), jnp.float32),
        pltpu.VMEM((BPW, E), jnp.float32),
        pltpu.SemaphoreType.DMA,
        pltpu.SemaphoreType.DMA,
        pltpu.SemaphoreType.DMA,
        pltpu.SemaphoreType.DMA,
        pltpu.SemaphoreType.DMA,
        pltpu.SemaphoreType.DMA,
        pltpu.SemaphoreType.DMA,
        pltpu.SemaphoreType.DMA,
    ],
    compiler_params=pltpu.CompilerParams(
        use_tc_tiling_on_sc=False, needs_layout_passes=False
    ),
)
def _pool(xq_hbm, xm_hbm, tab_hbm, out_hbm, xq_v, xm_v, rows_v, out_v,
          s0, s1, s2, s3, s4, s5, s6, s7):
    sems = (s0, s1, s2, s3, s4, s5, s6, s7)
    wid = lax.axis_index("s") * NC + lax.axis_index("c")
    base = wid * BPW
    pltpu.sync_copy(xq_hbm.at[pl.ds(base, BPW)], xq_v)
    pltpu.sync_copy(xm_hbm.at[pl.ds(base, BPW)], xm_v)

    for b in range(NBUF):
        pltpu.async_copy(tab_hbm.at[xq_v.at[b]], rows_v.at[b], sems[b])

    iota = lax.iota(jnp.int32, 16)

    def outer(g, _):
        r0 = g * NBUF
        for b in range(NBUF):
            r = r0 + b
            pltpu.make_async_copy(
                tab_hbm.at[pl.ds(0, S)], rows_v.at[b], sems[b]
            ).wait()
            rb = rows_v.at[b]
            a0 = jnp.zeros((16,), jnp.float32)
            a1 = jnp.zeros((16,), jnp.float32)
            rfull = jnp.full((16,), r, jnp.int32)
            for j in range(S):
                jfull = jnp.full((16,), j, jnp.int32)
                mb = plsc.load_gather(xm_v, [rfull, jfull])
                col0 = mb + iota
                a0 = a0 + plsc.load_gather(rb, [jfull, col0])
                a1 = a1 + plsc.load_gather(rb, [jfull, col0 + 16])
            out_v[r, pl.ds(0, 16)] = a0
            out_v[r, pl.ds(16, 16)] = a1

            nxt = r + NBUF

            @pl.when(nxt < BPW)
            def _():
                pltpu.async_copy(
                    tab_hbm.at[xq_v.at[nxt]], rows_v.at[b], sems[b]
                )

        return 0

    lax.fori_loop(0, BPW // NBUF, outer, 0)
    pltpu.sync_copy(out_v, out_hbm.at[pl.ds(wid * BPW, BPW)])


_TBLK = 8192
_NRB = 31
_RSTRIDE = _NRB * _TBLK


def _tr_body(in0, in1, in2, in3, out_ref):
    out_ref[...] = jnp.concatenate(
        [in0[...].T, in1[...].T, in2[...].T, in3[...].T], axis=1
    )


def _transpose_table(embT):
    last = (1000000 + _TBLK - 1) // _TBLK - 1
    specs = [
        pl.BlockSpec(
            (E, _TBLK),
            lambda i, q=q: (0, jnp.minimum(q * _NRB + i, last)),
        )
        for q in range(4)
    ]
    return pl.pallas_call(
        _tr_body,
        out_shape=jax.ShapeDtypeStruct((_RSTRIDE, 128), jnp.float32),
        grid=(_NRB,),
        in_specs=specs,
        out_specs=pl.BlockSpec((_TBLK, 128), lambda i: (i, 0)),
    )(embT, embT, embT, embT)


def _mlp_body(p_ref, w1_ref, b1_ref, w2_ref, b2_ref, o_ref):
    h = p_ref[...] * (1.0 / S)
    h = lax.dot_general(h, w1_ref[...], (((1,), (1,)), ((), ())),
                        preferred_element_type=jnp.float32)
    h = jnp.maximum(h + b1_ref[...], 0.0)
    o = lax.dot_general(h, w2_ref[...], (((1,), (1,)), ((), ())),
                        preferred_element_type=jnp.float32)
    o_ref[...] = o + b2_ref[...]


_BLK = 2048


def _mlp(pooled, W1, b1, W2, b2):
    grid = B // _BLK
    return pl.pallas_call(
        _mlp_body,
        out_shape=jax.ShapeDtypeStruct((B, NCLS), jnp.float32),
        grid=(grid,),
        in_specs=[
            pl.BlockSpec((_BLK, E), lambda i: (i, 0)),
            pl.BlockSpec((HID, E), lambda i: (0, 0)),
            pl.BlockSpec((1, HID), lambda i: (0, 0)),
            pl.BlockSpec((NCLS, HID), lambda i: (0, 0)),
            pl.BlockSpec((1, NCLS), lambda i: (0, 0)),
        ],
        out_specs=pl.BlockSpec((_BLK, NCLS), lambda i: (i, 0)),
    )(pooled, W1, b1, W2, b2)


def kernel(x, emb, W1, b1, W2, b2):
    table = _transpose_table(emb.T)
    q = x // _RSTRIDE
    xq = x - q * _RSTRIDE
    xm = q * 32
    pooled = _pool(xq, xm, table)
    return _mlp(pooled, W1, b1.reshape(1, HID), W2, b2.reshape(1, NCLS))


# speedup vs baseline: 1.3722x; 1.3722x over previous
import functools

import jax
import jax.numpy as jnp
from jax import lax
from jax.experimental import pallas as pl
from jax.experimental.pallas import tpu as pltpu
from jax.experimental.pallas import tpu_sc as plsc

B = 16384
S = 50
E = 32
HID = 32
NCLS = 10

NC = 2
NS = 16
NW = NC * NS
BPW = B // NW
NBUF = 8

_mesh = plsc.VectorSubcoreMesh(
    core_axis_name="c", subcore_axis_name="s", num_cores=NC, num_subcores=NS
)

_IOTA16 = tuple(range(16))


@functools.partial(
    pl.kernel,
    out_type=jax.ShapeDtypeStruct((B, E), jnp.float32),
    mesh=_mesh,
    scratch_types=[
        pltpu.VMEM((BPW, S), jnp.int32),
        pltpu.VMEM((BPW, S), jnp.int32),
        pltpu.VMEM((NBUF, S, 128), jnp.float32),
        pltpu.VMEM((BPW, E), jnp.float32),
        pltpu.SemaphoreType.DMA,
        pltpu.SemaphoreType.DMA,
        pltpu.SemaphoreType.DMA,
        pltpu.SemaphoreType.DMA,
        pltpu.SemaphoreType.DMA,
        pltpu.SemaphoreType.DMA,
        pltpu.SemaphoreType.DMA,
        pltpu.SemaphoreType.DMA,
    ],
    compiler_params=pltpu.CompilerParams(
        use_tc_tiling_on_sc=False, needs_layout_passes=False
    ),
)
def _pool(xq_hbm, xm_hbm, tab_hbm, out_hbm, xq_v, xm_v, rows_v, out_v,
          s0, s1, s2, s3, s4, s5, s6, s7):
    sems = (s0, s1, s2, s3, s4, s5, s6, s7)
    wid = lax.axis_index("s") * NC + lax.axis_index("c")
    base = wid * BPW
    pltpu.sync_copy(xq_hbm.at[pl.ds(base, BPW)], xq_v)
    pltpu.sync_copy(xm_hbm.at[pl.ds(base, BPW)], xm_v)

    for b in range(NBUF):
        pltpu.async_copy(tab_hbm.at[xq_v.at[b]], rows_v.at[b], sems[b])

    iota = lax.iota(jnp.int32, 16)

    def outer(g, _):
        r0 = g * NBUF
        for b in range(NBUF):
            r = r0 + b
            pltpu.make_async_copy(
                tab_hbm.at[pl.ds(0, S)], rows_v.at[b], sems[b]
            ).wait()
            rb = rows_v.at[b]
            a0 = jnp.zeros((16,), jnp.float32)
            a1 = jnp.zeros((16,), jnp.float32)
            rfull = jnp.full((16,), r, jnp.int32)
            for j in range(S):
                jfull = jnp.full((16,), j, jnp.int32)
                mb = plsc.load_gather(xm_v, [rfull, jfull])
                col0 = mb + iota
                a0 = a0 + plsc.load_gather(rb, [jfull, col0])
                a1 = a1 + plsc.load_gather(rb, [jfull, col0 + 16])
            out_v[r, pl.ds(0, 16)] = a0
            out_v[r, pl.ds(16, 16)] = a1

            nxt = r + NBUF

            @pl.when(nxt < BPW)
            def _():
                pltpu.async_copy(
                    tab_hbm.at[xq_v.at[nxt]], rows_v.at[b], sems[b]
                )

        return 0

    lax.fori_loop(0, BPW // NBUF, outer, 0)
    pltpu.sync_copy(out_v, out_hbm.at[pl.ds(wid * BPW, BPW)])


_TBLK = 4096
_NRB = 62
_RSTRIDE = _NRB * _TBLK


def _tr_body(in0, in1, in2, in3, out_ref):
    eye = jnp.eye(128, dtype=jnp.float32)
    stack = jnp.concatenate([in0[...], in1[...], in2[...], in3[...]], axis=0)
    out_ref[...] = lax.dot_general(
        stack, eye, (((0,), (0,)), ((), ())),
        preferred_element_type=jnp.float32,
    )


def _transpose_table(embT):
    last = (1000000 + _TBLK - 1) // _TBLK - 1
    specs = [
        pl.BlockSpec(
            (E, _TBLK),
            lambda i, q=q: (0, jnp.minimum(q * _NRB + i, last)),
        )
        for q in range(4)
    ]
    return pl.pallas_call(
        _tr_body,
        out_shape=jax.ShapeDtypeStruct((_RSTRIDE, 128), jnp.float32),
        grid=(_NRB,),
        in_specs=specs,
        out_specs=pl.BlockSpec((_TBLK, 128), lambda i: (i, 0)),
    )(embT, embT, embT, embT)


def _mlp_body(p_ref, w1_ref, b1_ref, w2_ref, b2_ref, o_ref):
    h = p_ref[...] * (1.0 / S)
    h = lax.dot_general(h, w1_ref[...], (((1,), (1,)), ((), ())),
                        preferred_element_type=jnp.float32)
    h = jnp.maximum(h + b1_ref[...], 0.0)
    o = lax.dot_general(h, w2_ref[...], (((1,), (1,)), ((), ())),
                        preferred_element_type=jnp.float32)
    o_ref[...] = o + b2_ref[...]


_BLK = 2048


def _mlp(pooled, W1, b1, W2, b2):
    grid = B // _BLK
    return pl.pallas_call(
        _mlp_body,
        out_shape=jax.ShapeDtypeStruct((B, NCLS), jnp.float32),
        grid=(grid,),
        in_specs=[
            pl.BlockSpec((_BLK, E), lambda i: (i, 0)),
            pl.BlockSpec((HID, E), lambda i: (0, 0)),
            pl.BlockSpec((1, HID), lambda i: (0, 0)),
            pl.BlockSpec((NCLS, HID), lambda i: (0, 0)),
            pl.BlockSpec((1, NCLS), lambda i: (0, 0)),
        ],
        out_specs=pl.BlockSpec((_BLK, NCLS), lambda i: (i, 0)),
    )(pooled, W1, b1, W2, b2)


def kernel(x, emb, W1, b1, W2, b2):
    table = _transpose_table(emb.T)
    q = x // _RSTRIDE
    xq = x - q * _RSTRIDE
    xm = q * 32
    pooled = _pool(xq, xm, table)
    return _mlp(pooled, W1, b1.reshape(1, HID), W2, b2.reshape(1, NCLS))


# speedup vs baseline: 1.4097x; 1.0273x over previous
import functools

import jax
import jax.numpy as jnp
from jax import lax
from jax.experimental import pallas as pl
from jax.experimental.pallas import tpu as pltpu
from jax.experimental.pallas import tpu_sc as plsc

B = 16384
S = 50
E = 32
HID = 32
NCLS = 10

NC = 2
NS = 16
NW = NC * NS
BPW = B // NW
NBUF = 8

_mesh = plsc.VectorSubcoreMesh(
    core_axis_name="c", subcore_axis_name="s", num_cores=NC, num_subcores=NS
)

_IOTA16 = tuple(range(16))


@functools.partial(
    pl.kernel,
    out_type=jax.ShapeDtypeStruct((B, E), jnp.float32),
    mesh=_mesh,
    scratch_types=[
        pltpu.VMEM((BPW, S), jnp.int32),
        pltpu.VMEM((BPW, S), jnp.int32),
        pltpu.VMEM((NBUF, S, 128), jnp.float32),
        pltpu.VMEM((BPW, E), jnp.float32),
        pltpu.SemaphoreType.DMA,
        pltpu.SemaphoreType.DMA,
        pltpu.SemaphoreType.DMA,
        pltpu.SemaphoreType.DMA,
        pltpu.SemaphoreType.DMA,
        pltpu.SemaphoreType.DMA,
        pltpu.SemaphoreType.DMA,
        pltpu.SemaphoreType.DMA,
    ],
    compiler_params=pltpu.CompilerParams(
        use_tc_tiling_on_sc=False, needs_layout_passes=False
    ),
)
def _pool(xqm_hbm, tab_hbm, out_hbm, xq_v, xm_v, rows_v, out_v,
          s0, s1, s2, s3, s4, s5, s6, s7):
    sems = (s0, s1, s2, s3, s4, s5, s6, s7)
    wid = lax.axis_index("s") * NC + lax.axis_index("c")
    base = wid * BPW
    pltpu.sync_copy(xqm_hbm.at[0, pl.ds(base, BPW)], xq_v)
    pltpu.sync_copy(xqm_hbm.at[1, pl.ds(base, BPW)], xm_v)

    for b in range(NBUF):
        pltpu.async_copy(tab_hbm.at[xq_v.at[b]], rows_v.at[b], sems[b])

    iota = lax.iota(jnp.int32, 16)

    def outer(g, _):
        r0 = g * NBUF
        for b in range(NBUF):
            r = r0 + b
            pltpu.make_async_copy(
                tab_hbm.at[pl.ds(0, S)], rows_v.at[b], sems[b]
            ).wait()
            rb = rows_v.at[b]
            a0 = jnp.zeros((16,), jnp.float32)
            a1 = jnp.zeros((16,), jnp.float32)
            rfull = jnp.full((16,), r, jnp.int32)
            for j in range(S):
                jfull = jnp.full((16,), j, jnp.int32)
                mb = plsc.load_gather(xm_v, [rfull, jfull])
                col0 = mb + iota
                a0 = a0 + plsc.load_gather(rb, [jfull, col0])
                a1 = a1 + plsc.load_gather(rb, [jfull, col0 + 16])
            out_v[r, pl.ds(0, 16)] = a0
            out_v[r, pl.ds(16, 16)] = a1

            nxt = r + NBUF

            @pl.when(nxt < BPW)
            def _():
                pltpu.async_copy(
                    tab_hbm.at[xq_v.at[nxt]], rows_v.at[b], sems[b]
                )

        return 0

    lax.fori_loop(0, BPW // NBUF, outer, 0)
    pltpu.sync_copy(out_v, out_hbm.at[pl.ds(wid * BPW, BPW)])


_TBLK = 4096
_NRB = 62
_RSTRIDE = _NRB * _TBLK


def _tr_body(in0, in1, in2, in3, out_ref):
    eye = jnp.eye(128, dtype=jnp.float32)
    stack = jnp.concatenate([in0[...], in1[...], in2[...], in3[...]], axis=0)
    out_ref[...] = lax.dot_general(
        stack, eye, (((0,), (0,)), ((), ())),
        preferred_element_type=jnp.float32,
    )


def _transpose_table(embT):
    last = (1000000 + _TBLK - 1) // _TBLK - 1
    specs = [
        pl.BlockSpec(
            (E, _TBLK),
            lambda i, q=q: (0, jnp.minimum(q * _NRB + i, last)),
        )
        for q in range(4)
    ]
    return pl.pallas_call(
        _tr_body,
        out_shape=jax.ShapeDtypeStruct((_RSTRIDE, 128), jnp.float32),
        grid=(_NRB,),
        in_specs=specs,
        out_specs=pl.BlockSpec((_TBLK, 128), lambda i: (i, 0)),
    )(embT, embT, embT, embT)


def _mlp_body(p_ref, w1_ref, b1_ref, w2_ref, b2_ref, o_ref):
    h = p_ref[...] * (1.0 / S)
    h = lax.dot_general(h, w1_ref[...], (((1,), (1,)), ((), ())),
                        preferred_element_type=jnp.float32)
    h = jnp.maximum(h + b1_ref[...], 0.0)
    o = lax.dot_general(h, w2_ref[...], (((1,), (1,)), ((), ())),
                        preferred_element_type=jnp.float32)
    o_ref[...] = o + b2_ref[...]


_BLK = 2048


def _mlp(pooled, W1, b1, W2, b2):
    grid = B // _BLK
    return pl.pallas_call(
        _mlp_body,
        out_shape=jax.ShapeDtypeStruct((B, NCLS), jnp.float32),
        grid=(grid,),
        in_specs=[
            pl.BlockSpec((_BLK, E), lambda i: (i, 0)),
            pl.BlockSpec((HID, E), lambda i: (0, 0)),
            pl.BlockSpec((1, HID), lambda i: (0, 0)),
            pl.BlockSpec((NCLS, HID), lambda i: (0, 0)),
            pl.BlockSpec((1, NCLS), lambda i: (0, 0)),
        ],
        out_specs=pl.BlockSpec((_BLK, NCLS), lambda i: (i, 0)),
    )(pooled, W1, b1, W2, b2)


def kernel(x, emb, W1, b1, W2, b2):
    table = _transpose_table(emb.T)
    q = x // _RSTRIDE
    xqm = jnp.stack([x - q * _RSTRIDE, q * 32])
    pooled = _pool(xqm, table)
    return _mlp(pooled, W1, b1.reshape(1, HID), W2, b2.reshape(1, NCLS))


# speedup vs baseline: 1.4126x; 1.0020x over previous
import functools

import jax
import jax.numpy as jnp
from jax import lax
from jax.experimental import pallas as pl
from jax.experimental.pallas import tpu as pltpu
from jax.experimental.pallas import tpu_sc as plsc

B = 16384
S = 50
E = 32
HID = 32
NCLS = 10

NC = 2
NS = 16
NW = NC * NS
BPW = B // NW
NBUF = 8

_mesh = plsc.VectorSubcoreMesh(
    core_axis_name="c", subcore_axis_name="s", num_cores=NC, num_subcores=NS
)

@functools.partial(
    pl.kernel,
    out_type=jax.ShapeDtypeStruct((B, E), jnp.float32),
    mesh=_mesh,
    scratch_types=[
        pltpu.VMEM((BPW, S), jnp.int32),
        pltpu.VMEM((BPW, S), jnp.int32),
        pltpu.VMEM((NBUF, S, 128), jnp.float32),
        pltpu.VMEM((BPW, E), jnp.float32),
        pltpu.SemaphoreType.DMA,
        pltpu.SemaphoreType.DMA,
        pltpu.SemaphoreType.DMA,
        pltpu.SemaphoreType.DMA,
        pltpu.SemaphoreType.DMA,
        pltpu.SemaphoreType.DMA,
        pltpu.SemaphoreType.DMA,
        pltpu.SemaphoreType.DMA,
    ],
    compiler_params=pltpu.CompilerParams(
        use_tc_tiling_on_sc=False, needs_layout_passes=False
    ),
)
def _pool(xqm_hbm, tab_hbm, out_hbm, xq_v, xm_v, rows_v, out_v,
          s0, s1, s2, s3, s4, s5, s6, s7):
    sems = (s0, s1, s2, s3, s4, s5, s6, s7)
    wid = lax.axis_index("s") * NC + lax.axis_index("c")
    base = wid * BPW
    pltpu.sync_copy(xqm_hbm.at[0, pl.ds(base, BPW)], xq_v)
    pltpu.sync_copy(xqm_hbm.at[1, pl.ds(base, BPW)], xm_v)

    for b in range(NBUF):
        pltpu.async_copy(tab_hbm.at[xq_v.at[b]], rows_v.at[b], sems[b])

    iota = lax.iota(jnp.int32, 16)

    def outer(g, _):
        r0 = g * NBUF
        for b in range(NBUF):
            r = r0 + b
            pltpu.make_async_copy(
                tab_hbm.at[pl.ds(0, S)], rows_v.at[b], sems[b]
            ).wait()
            rb = rows_v.at[b]
            a0 = jnp.zeros((16,), jnp.float32)
            a1 = jnp.zeros((16,), jnp.float32)
            rfull = jnp.full((16,), r, jnp.int32)
            for j in range(S):
                jfull = jnp.full((16,), j, jnp.int32)
                mb = plsc.load_gather(xm_v, [rfull, jfull])
                col0 = mb + iota
                a0 = a0 + plsc.load_gather(rb, [jfull, col0])
                a1 = a1 + plsc.load_gather(rb, [jfull, col0 + 16])
            out_v[r, pl.ds(0, 16)] = a0
            out_v[r, pl.ds(16, 16)] = a1

            nxt = r + NBUF

            @pl.when(nxt < BPW)
            def _():
                pltpu.async_copy(
                    tab_hbm.at[xq_v.at[nxt]], rows_v.at[b], sems[b]
                )

        return 0

    lax.fori_loop(0, BPW // NBUF, outer, 0)
    pltpu.sync_copy(out_v, out_hbm.at[pl.ds(wid * BPW, BPW)])


_TBLK = 4096
_NRB = 62
_RSTRIDE = _NRB * _TBLK


def _tr_body(in0, in1, in2, in3, out_ref):
    eye = jnp.eye(128, dtype=jnp.float32)
    stack = jnp.concatenate([in0[...], in1[...], in2[...], in3[...]], axis=0)
    out_ref[...] = lax.dot_general(
        stack, eye, (((0,), (0,)), ((), ())),
        preferred_element_type=jnp.float32,
    )


def _transpose_table(embT):
    last = (1000000 + _TBLK - 1) // _TBLK - 1
    specs = [
        pl.BlockSpec(
            (E, _TBLK),
            lambda i, q=q: (0, jnp.minimum(q * _NRB + i, last)),
        )
        for q in range(4)
    ]
    return pl.pallas_call(
        _tr_body,
        out_shape=jax.ShapeDtypeStruct((_RSTRIDE, 128), jnp.float32),
        grid=(_NRB,),
        in_specs=specs,
        out_specs=pl.BlockSpec((_TBLK, 128), lambda i: (i, 0)),
    )(embT, embT, embT, embT)


def _mlp_body(p_ref, w1_ref, b1_ref, w2_ref, b2_ref, o_ref):
    h = p_ref[...] * (1.0 / S)
    h = lax.dot_general(h, w1_ref[...], (((1,), (1,)), ((), ())),
                        preferred_element_type=jnp.float32)
    h = jnp.maximum(h + b1_ref[...], 0.0)
    o = lax.dot_general(h, w2_ref[...], (((1,), (1,)), ((), ())),
                        preferred_element_type=jnp.float32)
    o_ref[...] = o + b2_ref[...]


_BLK = 2048


def _mlp(pooled, W1, b1, W2, b2):
    grid = B // _BLK
    return pl.pallas_call(
        _mlp_body,
        out_shape=jax.ShapeDtypeStruct((B, NCLS), jnp.float32),
        grid=(grid,),
        in_specs=[
            pl.BlockSpec((_BLK, E), lambda i: (i, 0)),
            pl.BlockSpec((HID, E), lambda i: (0, 0)),
            pl.BlockSpec((1, HID), lambda i: (0, 0)),
            pl.BlockSpec((NCLS, HID), lambda i: (0, 0)),
            pl.BlockSpec((1, NCLS), lambda i: (0, 0)),
        ],
        out_specs=pl.BlockSpec((_BLK, NCLS), lambda i: (i, 0)),
    )(pooled, W1, b1, W2, b2)


def kernel(x, emb, W1, b1, W2, b2):
    table = _transpose_table(emb.T)
    q = x // _RSTRIDE
    xqm = jnp.stack([x - q * _RSTRIDE, q * 32])
    pooled = _pool(xqm, table)
    return _mlp(pooled, W1, b1.reshape(1, HID), W2, b2.reshape(1, NCLS))
